# baseline (speedup 1.0000x reference)
import functools
import math

import jax
import jax.numpy as jnp
from jax import lax
from jax.experimental import pallas as pl
from jax.experimental.pallas import tpu as pltpu
from jax.experimental.pallas import tpu_sc as plsc

_N = 10000
_E = 160000
_D = 256
_NL = 4
_ALPHA = 0.5

_NT = 32
_K = 128
_KB = 1024
_EPAD = 163840
_NSC = _EPAD // _KB
_RANGE = 320
_ROWS = _NT * _RANGE
_B = 64
_CAPB = _EPAD // _B + 4
_CAPW = _CAPB * 128
_TRASH = _RANGE

_BLK = 200
_NBLK = _N // _BLK

_sc_mesh = plsc.VectorSubcoreMesh(core_axis_name="c", subcore_axis_name="s")
_sc_params = pltpu.CompilerParams(needs_layout_passes=False)


@functools.partial(
    pl.kernel,
    out_type=(
        jax.ShapeDtypeStruct((_NT * _CAPW,), jnp.int32),
        jax.ShapeDtypeStruct((_NT, 16), jnp.int32),
        jax.ShapeDtypeStruct((_ROWS, 16), jnp.float32),
    ),
    mesh=_sc_mesh,
    compiler_params=_sc_params,
    scratch_types=[
        pltpu.VMEM((_KB,), jnp.int32),
        pltpu.VMEM((_KB,), jnp.int32),
        pltpu.VMEM((144,), jnp.int32),
        pltpu.VMEM((144,), jnp.int32),
        pltpu.VMEM((16,), jnp.int32),
        pltpu.VMEM((_RANGE + 1, 16), jnp.float32),
    ],
)
def _bucketize(row_hbm, col_hbm, bidx_out, cnt_out, deg_out,
               rbuf, cbuf, srow, slc, nvb, dacc):
    c = lax.axis_index("c")
    s = lax.axis_index("s")
    w = c * 16 + s
    obase = w * _CAPW
    lane = lax.iota(jnp.int32, 16)
    lo = w * _RANGE

    def _scan(t, carry):
        nf, cnt = carry
        pltpu.sync_copy(row_hbm.at[pl.ds(t * _KB, _KB)], rbuf)
        pltpu.sync_copy(col_hbm.at[pl.ds(t * _KB, _KB)], cbuf)

        def _grp(u, carry2):
            nf2, cnt2 = carry2
            rv = rbuf[pl.ds(u * 16, 16)]
            cv = cbuf[pl.ds(u * 16, 16)] - lo
            m = (cv >= 0) & (cv < _RANGE)
            mi = m.astype(jnp.int32)
            rank = plsc.cumsum(mi) - 1
            nm = jnp.sum(mi)
            off = jnp.where(m, nf2 + rank, 128 + lane)
            plsc.store_scatter(srow, [off], rv)
            plsc.store_scatter(slc, [off], cv)
            nf3 = nf2 + nm

            def _flush():
                base = obase + cnt2 * 128
                pltpu.sync_copy(srow.at[pl.ds(0, _B)],
                                bidx_out.at[pl.ds(base, _B)])
                pltpu.sync_copy(slc.at[pl.ds(0, _B)],
                                bidx_out.at[pl.ds(base + _B, _B)])
                srow[pl.ds(0, 16)] = srow[pl.ds(_B, 16)]
                slc[pl.ds(0, 16)] = slc[pl.ds(_B, 16)]

            pl.when(nf3 >= _B)(_flush)
            hit = nf3 >= _B
            return (jnp.where(hit, nf3 - _B, nf3),
                    jnp.where(hit, cnt2 + 1, cnt2))

        return lax.fori_loop(0, _KB // 16, _grp, (nf, cnt))

    nf, cnt = lax.fori_loop(0, _NSC, _scan, (jnp.int32(0), jnp.int32(0)))

    zero16 = jnp.zeros((16,), jnp.int32)
    trash16 = jnp.full((16,), _TRASH, jnp.int32)
    for t in range(5):
        plsc.store_scatter(srow, [nf + t * 16 + lane], zero16)
        plsc.store_scatter(slc, [nf + t * 16 + lane], trash16)
    base = obase + cnt * 128
    pltpu.sync_copy(srow.at[pl.ds(0, _B)], bidx_out.at[pl.ds(base, _B)])
    pltpu.sync_copy(slc.at[pl.ds(0, _B)], bidx_out.at[pl.ds(base + _B, _B)])
    nch = cnt + jnp.where(nf > 0, 1, 0).astype(jnp.int32)
    nvb[...] = jnp.broadcast_to(nch, (16,)).astype(jnp.int32)
    pltpu.sync_copy(nvb, cnt_out.at[w])

    fz = jnp.zeros((16,), jnp.float32)

    def _dz(i, carry):
        dacc[i, :] = fz
        return carry

    lax.fori_loop(0, _RANGE + 1, _dz, 0)
    one16 = jnp.ones((16,), jnp.float32)

    def _dchunk(j, carry):
        pltpu.sync_copy(bidx_out.at[pl.ds(obase + j * 128 + _B, _B)],
                        cbuf.at[pl.ds(0, _B)])
        for u in range(_B // 16):
            lcv = cbuf[pl.ds(u * 16, 16)]
            for e2 in range(16):
                lc = jnp.sum(jnp.where(lane == e2, lcv, 0))
                plsc.addupdate(dacc.at[lc], one16)
        return carry

    lax.fori_loop(0, nch, _dchunk, 0)
    pltpu.sync_copy(dacc.at[pl.ds(0, _RANGE)],
                    deg_out.at[pl.ds(w * _RANGE, _RANGE)])


@functools.partial(
    pl.kernel,
    out_type=jax.ShapeDtypeStruct((_ROWS, _D), jnp.float32),
    mesh=_sc_mesh,
    compiler_params=_sc_params,
    scratch_types=[
        pltpu.VMEM((16,), jnp.int32),
        pltpu.VMEM((128,), jnp.int32),
        pltpu.VMEM((128,), jnp.int32),
        pltpu.VMEM((_B, _D), jnp.float32),
        pltpu.VMEM((_B, _D), jnp.float32),
        pltpu.VMEM((_RANGE + 1, _D), jnp.float32),
        pltpu.SemaphoreType.DMA,
        pltpu.SemaphoreType.DMA,
    ],
)
def _propagate(bidx_hbm, cnt_hbm, g_hbm, s_out,
               nv, ib0, ib1, buf0, buf1, acc, sem0, sem1):
    c = lax.axis_index("c")
    s = lax.axis_index("s")
    w = c * 16 + s
    obase = w * _CAPW
    lane = lax.iota(jnp.int32, 16)

    pltpu.sync_copy(cnt_hbm.at[w], nv)
    n = jnp.sum(jnp.where(lane == 0, nv[...], 0))

    def _idx(j, ib):
        pltpu.sync_copy(bidx_hbm.at[pl.ds(obase + j * 128, 128)], ib)

    def _fire(ib, buf, sem):
        pltpu.async_copy(g_hbm.at[ib.at[pl.ds(0, _B)]], buf, sem)

    def _wait(ib, buf, sem):
        pltpu.make_async_copy(g_hbm.at[ib.at[pl.ds(0, _B)]], buf, sem).wait()

    def _compute(ib, buf):
        for u in range(_B // 16):
            lcv = ib[pl.ds(_B + u * 16, 16)]
            rowv = lane + u * 16

            def _pp(d, colv):
                for _q in range(16):
                    vals = plsc.load_gather(buf, [rowv, colv])
                    plsc.addupdate_scatter(acc, [lcv, colv], vals)
                    colv = colv + 1
                return colv

            lax.fori_loop(0, _D // 16, _pp, jnp.zeros((16,), jnp.int32))

    @pl.when(n > 0)
    def _prime():
        _idx(0, ib0)
        _fire(ib0, buf0, sem0)

    fz = jnp.zeros((16,), jnp.float32)

    def _az(i, carry):
        for v in range(_D // 16):
            acc[i, pl.ds(v * 16, 16)] = fz
        return carry

    lax.fori_loop(0, _RANGE + 1, _az, 0)

    def _body(jj, carry):
        j0 = jj * 2
        j1 = j0 + 1

        @pl.when(j1 < n)
        def _():
            _idx(j1, ib1)
            _fire(ib1, buf1, sem1)

        _wait(ib0, buf0, sem0)
        _compute(ib0, buf0)

        @pl.when(j0 + 2 < n)
        def _():
            _idx(j0 + 2, ib0)
            _fire(ib0, buf0, sem0)

        @pl.when(j1 < n)
        def _():
            _wait(ib1, buf1, sem1)
            _compute(ib1, buf1)

        return carry

    lax.fori_loop(0, (n + 1) // 2, _body, 0)
    pltpu.sync_copy(acc.at[pl.ds(0, _RANGE)],
                    s_out.at[pl.ds(w * _RANGE, _RANGE)])


def _prologue_body(x_ref, wp_ref, bp_ref, w2_ref, deg_ref,
                   g_ref, c_ref, dis_ref):
    h0 = jnp.dot(x_ref[...], wp_ref[...],
                 preferred_element_type=jnp.float32) + bp_ref[...]
    dis = lax.rsqrt(deg_ref[...][:, 0:1] + 1.0)
    g_ref[...] = dis * h0
    dis_ref[...] = jnp.broadcast_to(dis, (_BLK, 128))
    for i in range(_NL):
        beta = math.log(1.0 / (i + 1) + 1.0)
        c_ref[i, :, :] = (_ALPHA * (1.0 - beta)) * h0 + (_ALPHA * beta) * jnp.dot(
            h0, w2_ref[i], preferred_element_type=jnp.float32)


_prologue = pl.pallas_call(
    _prologue_body,
    grid=(_NBLK,),
    in_specs=[
        pl.BlockSpec((_BLK, _D), lambda j: (j, 0)),
        pl.BlockSpec((_D, _D), lambda j: (0, 0)),
        pl.BlockSpec((1, _D), lambda j: (0, 0)),
        pl.BlockSpec((_NL, _D, _D), lambda j: (0, 0, 0)),
        pl.BlockSpec((_BLK, 16), lambda j: (j, 0)),
    ],
    out_specs=[
        pl.BlockSpec((_BLK, _D), lambda j: (j, 0)),
        pl.BlockSpec((_NL, _BLK, _D), lambda j: (0, j, 0)),
        pl.BlockSpec((_BLK, 128), lambda j: (j, 0)),
    ],
    out_shape=[
        jax.ShapeDtypeStruct((_N, _D), jnp.float32),
        jax.ShapeDtypeStruct((_NL, _N, _D), jnp.float32),
        jax.ShapeDtypeStruct((_N, 128), jnp.float32),
    ],
)


def _layer_body(beta, is_last, s_ref, g_ref, cc_ref, dis_ref, w1_ref, o_ref):
    dis = dis_ref[...][:, 0:1]
    xp = dis * (s_ref[...] + g_ref[...])
    x1 = _ALPHA * xp
    out = cc_ref[...] + (1.0 - beta) * x1 + beta * jnp.dot(
        x1, w1_ref[...], preferred_element_type=jnp.float32)
    h = jnp.maximum(out, 0.0)
    o_ref[...] = h if is_last else dis * h


def _make_layer(i):
    beta = math.log(1.0 / (i + 1) + 1.0)
    return pl.pallas_call(
        functools.partial(_layer_body, beta, i == _NL - 1),
        grid=(_NBLK,),
        in_specs=[
            pl.BlockSpec((_BLK, _D), lambda j: (j, 0)),
            pl.BlockSpec((_BLK, _D), lambda j: (j, 0)),
            pl.BlockSpec((_BLK, _D), lambda j: (j, 0)),
            pl.BlockSpec((_BLK, 128), lambda j: (j, 0)),
            pl.BlockSpec((_D, _D), lambda j: (0, 0)),
        ],
        out_specs=pl.BlockSpec((_BLK, _D), lambda j: (j, 0)),
        out_shape=jax.ShapeDtypeStruct((_N, _D), jnp.float32),
    )


_layers = [_make_layer(i) for i in range(_NL)]


def kernel(x, edge_index, Wp, bp, W1, W2):
    row = edge_index[0]
    col = edge_index[1]
    pad = _EPAD - _E
    rowp = jnp.concatenate([row, jnp.zeros((pad,), row.dtype)])
    colp = jnp.concatenate([col, jnp.full((pad,), _ROWS, col.dtype)])

    bidx, bcnt, deg = _bucketize(rowp, colp)
    g, C, dis = _prologue(x, Wp, bp.reshape(1, _D), W2, deg[:_N])

    for i in range(_NL):
        spart = _propagate(bidx, bcnt, g)
        g = _layers[i](spart, g, C[i], dis, W1[i])
    return g


# speedup vs baseline: 2.9948x; 2.9948x over previous
import functools
import math

import jax
import jax.numpy as jnp
from jax import lax
from jax.experimental import pallas as pl
from jax.experimental.pallas import tpu as pltpu
from jax.experimental.pallas import tpu_sc as plsc

_N = 10000
_E = 160000
_D = 256
_NL = 4
_ALPHA = 0.5

_NT = 32
_K = 128
_KB = 1024
_EPAD = 163840
_NSC = _EPAD // _KB
_RANGE = 320
_ROWS = _NT * _RANGE
_B = 64
_CAPB = _EPAD // _B + 4
_CAPW = _CAPB * 128
_TRASH = _RANGE

_BLK = 200
_NBLK = _N // _BLK

_sc_mesh = plsc.VectorSubcoreMesh(core_axis_name="c", subcore_axis_name="s")
_sc_params = pltpu.CompilerParams(needs_layout_passes=False)


@functools.partial(
    pl.kernel,
    out_type=(
        jax.ShapeDtypeStruct((_NT * _CAPW,), jnp.int32),
        jax.ShapeDtypeStruct((_NT, 16), jnp.int32),
        jax.ShapeDtypeStruct((_ROWS, 16), jnp.float32),
    ),
    mesh=_sc_mesh,
    compiler_params=_sc_params,
    scratch_types=[
        pltpu.VMEM((_KB,), jnp.int32),
        pltpu.VMEM((_KB,), jnp.int32),
        pltpu.VMEM((144,), jnp.int32),
        pltpu.VMEM((144,), jnp.int32),
        pltpu.VMEM((16,), jnp.int32),
        pltpu.VMEM((_RANGE + 1, 16), jnp.float32),
    ],
)
def _bucketize(row_hbm, col_hbm, bidx_out, cnt_out, deg_out,
               rbuf, cbuf, srow, slc, nvb, dacc):
    c = lax.axis_index("c")
    s = lax.axis_index("s")
    w = c * 16 + s
    obase = w * _CAPW
    lane = lax.iota(jnp.int32, 16)
    lo = w * _RANGE

    def _scan(t, carry):
        nf, cnt = carry
        pltpu.sync_copy(row_hbm.at[pl.ds(t * _KB, _KB)], rbuf)
        pltpu.sync_copy(col_hbm.at[pl.ds(t * _KB, _KB)], cbuf)

        def _grp(u, carry2):
            nf2, cnt2 = carry2
            rv = rbuf[pl.ds(u * 16, 16)]
            cv = cbuf[pl.ds(u * 16, 16)] - lo
            m = (cv >= 0) & (cv < _RANGE)
            mi = m.astype(jnp.int32)
            rank = plsc.cumsum(mi) - 1
            nm = jnp.sum(mi)
            off = jnp.where(m, nf2 + rank, 128 + lane)
            plsc.store_scatter(srow, [off], rv)
            plsc.store_scatter(slc, [off], cv)
            nf3 = nf2 + nm

            def _flush():
                base = obase + cnt2 * 128
                pltpu.sync_copy(srow.at[pl.ds(0, _B)],
                                bidx_out.at[pl.ds(base, _B)])
                pltpu.sync_copy(slc.at[pl.ds(0, _B)],
                                bidx_out.at[pl.ds(base + _B, _B)])
                srow[pl.ds(0, 16)] = srow[pl.ds(_B, 16)]
                slc[pl.ds(0, 16)] = slc[pl.ds(_B, 16)]

            pl.when(nf3 >= _B)(_flush)
            hit = nf3 >= _B
            return (jnp.where(hit, nf3 - _B, nf3),
                    jnp.where(hit, cnt2 + 1, cnt2))

        return lax.fori_loop(0, _KB // 16, _grp, (nf, cnt))

    nf, cnt = lax.fori_loop(0, _NSC, _scan, (jnp.int32(0), jnp.int32(0)))

    zero16 = jnp.zeros((16,), jnp.int32)
    trash16 = jnp.full((16,), _TRASH, jnp.int32)
    for t in range(5):
        plsc.store_scatter(srow, [nf + t * 16 + lane], zero16)
        plsc.store_scatter(slc, [nf + t * 16 + lane], trash16)
    base = obase + cnt * 128
    pltpu.sync_copy(srow.at[pl.ds(0, _B)], bidx_out.at[pl.ds(base, _B)])
    pltpu.sync_copy(slc.at[pl.ds(0, _B)], bidx_out.at[pl.ds(base + _B, _B)])
    nch = cnt + jnp.where(nf > 0, 1, 0).astype(jnp.int32)
    nvb[...] = jnp.broadcast_to(nch, (16,)).astype(jnp.int32)
    pltpu.sync_copy(nvb, cnt_out.at[w])

    fz = jnp.zeros((16,), jnp.float32)

    def _dz(i, carry):
        dacc[i, :] = fz
        return carry

    lax.fori_loop(0, _RANGE + 1, _dz, 0)
    one16 = jnp.ones((16,), jnp.float32)

    def _dchunk(j, carry):
        pltpu.sync_copy(bidx_out.at[pl.ds(obase + j * 128 + _B, _B)],
                        cbuf.at[pl.ds(0, _B)])
        for u in range(_B // 16):
            lcv = cbuf[pl.ds(u * 16, 16)]
            for e2 in range(16):
                lc = jnp.sum(jnp.where(lane == e2, lcv, 0))
                plsc.addupdate(dacc.at[lc], one16)
        return carry

    lax.fori_loop(0, nch, _dchunk, 0)
    pltpu.sync_copy(dacc.at[pl.ds(0, _RANGE)],
                    deg_out.at[pl.ds(w * _RANGE, _RANGE)])


@functools.partial(
    pl.kernel,
    out_type=jax.ShapeDtypeStruct((_ROWS * _D,), jnp.float32),
    mesh=_sc_mesh,
    compiler_params=_sc_params,
    scratch_types=[
        pltpu.VMEM((16,), jnp.int32),
        pltpu.VMEM((128,), jnp.int32),
        pltpu.VMEM((128,), jnp.int32),
        pltpu.VMEM((_B, _D), jnp.float32),
        pltpu.VMEM((_B, _D), jnp.float32),
        pltpu.VMEM(((_RANGE + 1) * _D,), jnp.float32),
        pltpu.SemaphoreType.DMA,
        pltpu.SemaphoreType.DMA,
    ],
)
def _propagate(bidx_hbm, cnt_hbm, g_hbm, s_out,
               nv, ib0, ib1, buf0, buf1, acc, sem0, sem1):
    c = lax.axis_index("c")
    s = lax.axis_index("s")
    w = c * 16 + s
    obase = w * _CAPW
    lane = lax.iota(jnp.int32, 16)

    pltpu.sync_copy(cnt_hbm.at[w], nv)
    n = jnp.sum(jnp.where(lane == 0, nv[...], 0))

    def _idx(j, ib):
        pltpu.sync_copy(bidx_hbm.at[pl.ds(obase + j * 128, 128)], ib)

    def _fire(ib, buf, sem):
        pltpu.async_copy(g_hbm.at[ib.at[pl.ds(0, _B)]], buf, sem)

    def _wait(ib, buf, sem):
        pltpu.make_async_copy(g_hbm.at[ib.at[pl.ds(0, _B)]], buf, sem).wait()

    def _compute(ib, buf):
        for u in range(_B // 16):
            lcv = ib[pl.ds(_B + u * 16, 16)]
            for e2 in range(16):
                lcb = lcv.at[jnp.full((16,), e2, jnp.int32)].get(
                    mode="promise_in_bounds")
                tgt = lcb * _D + lane
                e = u * 16 + e2
                for d in range(_D // 16):
                    plsc.addupdate_scatter(acc, [tgt + d * 16],
                                           buf[e, pl.ds(d * 16, 16)])

    @pl.when(n > 0)
    def _prime():
        _idx(0, ib0)
        _fire(ib0, buf0, sem0)

    fz = jnp.zeros((16,), jnp.float32)

    def _az(i, carry):
        acc[pl.ds(i * 16, 16)] = fz
        return carry

    lax.fori_loop(0, (_RANGE + 1) * _D // 16, _az, 0)

    def _body(jj, carry):
        j0 = jj * 2
        j1 = j0 + 1

        @pl.when(j1 < n)
        def _():
            _idx(j1, ib1)
            _fire(ib1, buf1, sem1)

        _wait(ib0, buf0, sem0)
        _compute(ib0, buf0)

        @pl.when(j0 + 2 < n)
        def _():
            _idx(j0 + 2, ib0)
            _fire(ib0, buf0, sem0)

        @pl.when(j1 < n)
        def _():
            _wait(ib1, buf1, sem1)
            _compute(ib1, buf1)

        return carry

    lax.fori_loop(0, (n + 1) // 2, _body, 0)
    pltpu.sync_copy(acc.at[pl.ds(0, _RANGE * _D)],
                    s_out.at[pl.ds(w * _RANGE * _D, _RANGE * _D)])


def _prologue_body(x_ref, wp_ref, bp_ref, w2_ref, deg_ref,
                   g_ref, c_ref, dis_ref):
    h0 = jnp.dot(x_ref[...], wp_ref[...],
                 preferred_element_type=jnp.float32) + bp_ref[...]
    dis = lax.rsqrt(deg_ref[...][:, 0:1] + 1.0)
    g_ref[...] = dis * h0
    dis_ref[...] = jnp.broadcast_to(dis, (_BLK, 128))
    for i in range(_NL):
        beta = math.log(1.0 / (i + 1) + 1.0)
        c_ref[i, :, :] = (_ALPHA * (1.0 - beta)) * h0 + (_ALPHA * beta) * jnp.dot(
            h0, w2_ref[i], preferred_element_type=jnp.float32)


_prologue = pl.pallas_call(
    _prologue_body,
    grid=(_NBLK,),
    in_specs=[
        pl.BlockSpec((_BLK, _D), lambda j: (j, 0)),
        pl.BlockSpec((_D, _D), lambda j: (0, 0)),
        pl.BlockSpec((1, _D), lambda j: (0, 0)),
        pl.BlockSpec((_NL, _D, _D), lambda j: (0, 0, 0)),
        pl.BlockSpec((_BLK, 16), lambda j: (j, 0)),
    ],
    out_specs=[
        pl.BlockSpec((_BLK, _D), lambda j: (j, 0)),
        pl.BlockSpec((_NL, _BLK, _D), lambda j: (0, j, 0)),
        pl.BlockSpec((_BLK, 128), lambda j: (j, 0)),
    ],
    out_shape=[
        jax.ShapeDtypeStruct((_N, _D), jnp.float32),
        jax.ShapeDtypeStruct((_NL, _N, _D), jnp.float32),
        jax.ShapeDtypeStruct((_N, 128), jnp.float32),
    ],
)


def _layer_body(beta, is_last, s_ref, g_ref, cc_ref, dis_ref, w1_ref, o_ref):
    dis = dis_ref[...][:, 0:1]
    xp = dis * (s_ref[...] + g_ref[...])
    x1 = _ALPHA * xp
    out = cc_ref[...] + (1.0 - beta) * x1 + beta * jnp.dot(
        x1, w1_ref[...], preferred_element_type=jnp.float32)
    h = jnp.maximum(out, 0.0)
    o_ref[...] = h if is_last else dis * h


def _make_layer(i):
    beta = math.log(1.0 / (i + 1) + 1.0)
    return pl.pallas_call(
        functools.partial(_layer_body, beta, i == _NL - 1),
        grid=(_NBLK,),
        in_specs=[
            pl.BlockSpec((_BLK, _D), lambda j: (j, 0)),
            pl.BlockSpec((_BLK, _D), lambda j: (j, 0)),
            pl.BlockSpec((_BLK, _D), lambda j: (j, 0)),
            pl.BlockSpec((_BLK, 128), lambda j: (j, 0)),
            pl.BlockSpec((_D, _D), lambda j: (0, 0)),
        ],
        out_specs=pl.BlockSpec((_BLK, _D), lambda j: (j, 0)),
        out_shape=jax.ShapeDtypeStruct((_N, _D), jnp.float32),
    )


_layers = [_make_layer(i) for i in range(_NL)]


def kernel(x, edge_index, Wp, bp, W1, W2):
    row = edge_index[0]
    col = edge_index[1]
    pad = _EPAD - _E
    rowp = jnp.concatenate([row, jnp.zeros((pad,), row.dtype)])
    colp = jnp.concatenate([col, jnp.full((pad,), _ROWS, col.dtype)])

    bidx, bcnt, deg = _bucketize(rowp, colp)
    g, C, dis = _prologue(x, Wp, bp.reshape(1, _D), W2, deg[:_N])

    for i in range(_NL):
        spart = _propagate(bidx, bcnt, g).reshape(_ROWS, _D)
        g = _layers[i](spart, g, C[i], dis, W1[i])
    return g


# speedup vs baseline: 3.2210x; 1.0755x over previous
import functools
import math

import jax
import jax.numpy as jnp
from jax import lax
from jax.experimental import pallas as pl
from jax.experimental.pallas import tpu as pltpu
from jax.experimental.pallas import tpu_sc as plsc

_N = 10000
_E = 160000
_D = 256
_NL = 4
_ALPHA = 0.5

_NT = 32
_K = 128
_KB = 1024
_EPAD = 163840
_NSC = _EPAD // _KB
_RANGE = 320
_ROWS = _NT * _RANGE
_B = 64
_CAPB = _EPAD // _B + 4
_CAPW = _CAPB * 128
_TRASH = _RANGE

_BLK = 200
_NBLK = _N // _BLK

_sc_mesh = plsc.VectorSubcoreMesh(core_axis_name="c", subcore_axis_name="s")
_sc_params = pltpu.CompilerParams(needs_layout_passes=False)


@functools.partial(
    pl.kernel,
    out_type=(
        jax.ShapeDtypeStruct((_NT * _CAPW,), jnp.int32),
        jax.ShapeDtypeStruct((_NT, 16), jnp.int32),
        jax.ShapeDtypeStruct((_ROWS, 16), jnp.float32),
    ),
    mesh=_sc_mesh,
    compiler_params=_sc_params,
    scratch_types=[
        pltpu.VMEM((2 * _KB,), jnp.int32),
        pltpu.VMEM((2 * _KB,), jnp.int32),
        pltpu.VMEM((144,), jnp.int32),
        pltpu.VMEM((144,), jnp.int32),
        pltpu.VMEM((16,), jnp.int32),
        pltpu.VMEM((_RANGE + 1, 16), jnp.float32),
        pltpu.SemaphoreType.DMA,
        pltpu.SemaphoreType.DMA,
    ],
)
def _bucketize(eint_hbm, bidx_out, cnt_out, deg_out,
               ebuf0, ebuf1, srow, slc, nvb, dacc, semA, semB):
    c = lax.axis_index("c")
    s = lax.axis_index("s")
    w = c * 16 + s
    obase = w * _CAPW
    lane = lax.iota(jnp.int32, 16)
    lo = w * _RANGE

    def _fire(t, ebuf, sem):
        pltpu.async_copy(eint_hbm.at[pl.ds(t * 2 * _KB, 2 * _KB)], ebuf, sem)

    def _waitdma(t, ebuf, sem):
        pltpu.make_async_copy(eint_hbm.at[pl.ds(t * 2 * _KB, 2 * _KB)],
                              ebuf, sem).wait()

    def _scanbuf(ebuf, carry):
        def _grp(u, carry2):
            nf2, cnt2 = carry2
            rv = ebuf[pl.ds(u * 16, 16)]
            cv = ebuf[pl.ds(_KB + u * 16, 16)] - lo
            m = (cv >= 0) & (cv < _RANGE)
            mi = m.astype(jnp.int32)
            rank = plsc.cumsum(mi) - 1
            nm = jnp.sum(mi)
            off = jnp.where(m, nf2 + rank, 128 + lane)
            plsc.store_scatter(srow, [off], rv)
            plsc.store_scatter(slc, [off], cv)
            nf3 = nf2 + nm

            def _flush():
                base = obase + cnt2 * 128
                pltpu.sync_copy(srow.at[pl.ds(0, _B)],
                                bidx_out.at[pl.ds(base, _B)])
                pltpu.sync_copy(slc.at[pl.ds(0, _B)],
                                bidx_out.at[pl.ds(base + _B, _B)])
                srow[pl.ds(0, 16)] = srow[pl.ds(_B, 16)]
                slc[pl.ds(0, 16)] = slc[pl.ds(_B, 16)]

            pl.when(nf3 >= _B)(_flush)
            hit = nf3 >= _B
            return (jnp.where(hit, nf3 - _B, nf3),
                    jnp.where(hit, cnt2 + 1, cnt2))

        return lax.fori_loop(0, _KB // 16, _grp, carry)

    _fire(0, ebuf0, semA)

    def _scan2(tt, carry):
        t0 = tt * 2
        t1 = t0 + 1
        _fire(t1, ebuf1, semB)
        _waitdma(t0, ebuf0, semA)
        carry = _scanbuf(ebuf0, carry)

        @pl.when(t0 + 2 < _NSC)
        def _():
            _fire(t0 + 2, ebuf0, semA)

        _waitdma(t1, ebuf1, semB)
        return _scanbuf(ebuf1, carry)

    nf, cnt = lax.fori_loop(0, _NSC // 2, _scan2,
                            (jnp.int32(0), jnp.int32(0)))

    zero16 = jnp.zeros((16,), jnp.int32)
    trash16 = jnp.full((16,), _TRASH, jnp.int32)
    for t in range(5):
        plsc.store_scatter(srow, [nf + t * 16 + lane], zero16)
        plsc.store_scatter(slc, [nf + t * 16 + lane], trash16)
    base = obase + cnt * 128
    pltpu.sync_copy(srow.at[pl.ds(0, _B)], bidx_out.at[pl.ds(base, _B)])
    pltpu.sync_copy(slc.at[pl.ds(0, _B)], bidx_out.at[pl.ds(base + _B, _B)])
    nch = cnt + jnp.where(nf > 0, 1, 0).astype(jnp.int32)
    nvb[...] = jnp.broadcast_to(nch, (16,)).astype(jnp.int32)
    pltpu.sync_copy(nvb, cnt_out.at[w])

    fz = jnp.zeros((16,), jnp.float32)

    def _dz(i, carry):
        dacc[i, :] = fz
        return carry

    lax.fori_loop(0, _RANGE + 1, _dz, 0)
    one16 = jnp.ones((16,), jnp.float32)

    def _dchunk(j, carry):
        pltpu.sync_copy(bidx_out.at[pl.ds(obase + j * 128 + _B, _B)],
                        ebuf0.at[pl.ds(0, _B)])
        for u in range(_B // 16):
            lcv = ebuf0[pl.ds(u * 16, 16)]
            for e2 in range(16):
                lc = jnp.sum(jnp.where(lane == e2, lcv, 0))
                plsc.addupdate(dacc.at[lc], one16)
        return carry

    lax.fori_loop(0, nch, _dchunk, 0)
    pltpu.sync_copy(dacc.at[pl.ds(0, _RANGE)],
                    deg_out.at[pl.ds(w * _RANGE, _RANGE)])


@functools.partial(
    pl.kernel,
    out_type=jax.ShapeDtypeStruct((_ROWS * _D,), jnp.float32),
    mesh=_sc_mesh,
    compiler_params=_sc_params,
    scratch_types=[
        pltpu.VMEM((16,), jnp.int32),
        pltpu.VMEM((128,), jnp.int32),
        pltpu.VMEM((128,), jnp.int32),
        pltpu.VMEM((_B, _D), jnp.float32),
        pltpu.VMEM((_B, _D), jnp.float32),
        pltpu.VMEM(((_RANGE + 1) * _D,), jnp.float32),
        pltpu.SemaphoreType.DMA,
        pltpu.SemaphoreType.DMA,
    ],
)
def _propagate(bidx_hbm, cnt_hbm, g_hbm, s_out,
               nv, ib0, ib1, buf0, buf1, acc, sem0, sem1):
    c = lax.axis_index("c")
    s = lax.axis_index("s")
    w = c * 16 + s
    obase = w * _CAPW
    lane = lax.iota(jnp.int32, 16)

    pltpu.sync_copy(cnt_hbm.at[w], nv)
    n = jnp.sum(jnp.where(lane == 0, nv[...], 0))

    def _idx(j, ib):
        pltpu.sync_copy(bidx_hbm.at[pl.ds(obase + j * 128, 128)], ib)

    def _fire(ib, buf, sem):
        pltpu.async_copy(g_hbm.at[ib.at[pl.ds(0, _B)]], buf, sem)

    def _wait(ib, buf, sem):
        pltpu.make_async_copy(g_hbm.at[ib.at[pl.ds(0, _B)]], buf, sem).wait()

    def _compute(ib, buf):
        for u in range(_B // 16):
            lcv = ib[pl.ds(_B + u * 16, 16)]
            for e2 in range(16):
                lcb = lcv.at[jnp.full((16,), e2, jnp.int32)].get(
                    mode="promise_in_bounds")
                tgt = lcb * _D + lane
                e = u * 16 + e2
                for d in range(_D // 16):
                    plsc.addupdate_scatter(acc, [tgt + d * 16],
                                           buf[e, pl.ds(d * 16, 16)])

    @pl.when(n > 0)
    def _prime():
        _idx(0, ib0)
        _fire(ib0, buf0, sem0)

    fz = jnp.zeros((16,), jnp.float32)

    def _az(i, carry):
        acc[pl.ds(i * 16, 16)] = fz
        return carry

    lax.fori_loop(0, (_RANGE + 1) * _D // 16, _az, 0)

    def _body(jj, carry):
        j0 = jj * 2
        j1 = j0 + 1

        @pl.when(j1 < n)
        def _():
            _idx(j1, ib1)
            _fire(ib1, buf1, sem1)

        _wait(ib0, buf0, sem0)
        _compute(ib0, buf0)

        @pl.when(j0 + 2 < n)
        def _():
            _idx(j0 + 2, ib0)
            _fire(ib0, buf0, sem0)

        @pl.when(j1 < n)
        def _():
            _wait(ib1, buf1, sem1)
            _compute(ib1, buf1)

        return carry

    lax.fori_loop(0, (n + 1) // 2, _body, 0)
    pltpu.sync_copy(acc.at[pl.ds(0, _RANGE * _D)],
                    s_out.at[pl.ds(w * _RANGE * _D, _RANGE * _D)])


def _prologue_body(x_ref, wp_ref, bp_ref, w2_ref, deg_ref,
                   g_ref, c_ref, dis_ref):
    h0 = jnp.dot(x_ref[...], wp_ref[...],
                 preferred_element_type=jnp.float32) + bp_ref[...]
    dis = lax.rsqrt(deg_ref[...][:, 0:1] + 1.0)
    g_ref[...] = dis * h0
    dis_ref[...] = jnp.broadcast_to(dis, (_BLK, 128))
    for i in range(_NL):
        beta = math.log(1.0 / (i + 1) + 1.0)
        c_ref[i, :, :] = (_ALPHA * (1.0 - beta)) * h0 + (_ALPHA * beta) * jnp.dot(
            h0, w2_ref[i], preferred_element_type=jnp.float32)


_prologue = pl.pallas_call(
    _prologue_body,
    grid=(_NBLK,),
    in_specs=[
        pl.BlockSpec((_BLK, _D), lambda j: (j, 0)),
        pl.BlockSpec((_D, _D), lambda j: (0, 0)),
        pl.BlockSpec((1, _D), lambda j: (0, 0)),
        pl.BlockSpec((_NL, _D, _D), lambda j: (0, 0, 0)),
        pl.BlockSpec((_BLK, 16), lambda j: (j, 0)),
    ],
    out_specs=[
        pl.BlockSpec((_BLK, _D), lambda j: (j, 0)),
        pl.BlockSpec((_NL, _BLK, _D), lambda j: (0, j, 0)),
        pl.BlockSpec((_BLK, 128), lambda j: (j, 0)),
    ],
    out_shape=[
        jax.ShapeDtypeStruct((_N, _D), jnp.float32),
        jax.ShapeDtypeStruct((_NL, _N, _D), jnp.float32),
        jax.ShapeDtypeStruct((_N, 128), jnp.float32),
    ],
)


def _layer_body(beta, is_last, s_ref, g_ref, cc_ref, dis_ref, w1_ref, o_ref):
    dis = dis_ref[...][:, 0:1]
    xp = dis * (s_ref[...] + g_ref[...])
    x1 = _ALPHA * xp
    out = cc_ref[...] + (1.0 - beta) * x1 + beta * jnp.dot(
        x1, w1_ref[...], preferred_element_type=jnp.float32)
    h = jnp.maximum(out, 0.0)
    o_ref[...] = h if is_last else dis * h


def _make_layer(i):
    beta = math.log(1.0 / (i + 1) + 1.0)
    return pl.pallas_call(
        functools.partial(_layer_body, beta, i == _NL - 1),
        grid=(_NBLK,),
        in_specs=[
            pl.BlockSpec((_BLK, _D), lambda j: (j, 0)),
            pl.BlockSpec((_BLK, _D), lambda j: (j, 0)),
            pl.BlockSpec((_BLK, _D), lambda j: (j, 0)),
            pl.BlockSpec((_BLK, 128), lambda j: (j, 0)),
            pl.BlockSpec((_D, _D), lambda j: (0, 0)),
        ],
        out_specs=pl.BlockSpec((_BLK, _D), lambda j: (j, 0)),
        out_shape=jax.ShapeDtypeStruct((_N, _D), jnp.float32),
    )


_layers = [_make_layer(i) for i in range(_NL)]


def kernel(x, edge_index, Wp, bp, W1, W2):
    row = edge_index[0]
    col = edge_index[1]
    pad = _EPAD - _E
    rowp = jnp.concatenate([row, jnp.zeros((pad,), row.dtype)])
    colp = jnp.concatenate([col, jnp.full((pad,), _ROWS, col.dtype)])
    eint = jnp.concatenate([rowp.reshape(_NSC, _KB),
                            colp.reshape(_NSC, _KB)], axis=1).reshape(-1)

    bidx, bcnt, deg = _bucketize(eint)
    g, C, dis = _prologue(x, Wp, bp.reshape(1, _D), W2, deg[:_N])

    for i in range(_NL):
        spart = _propagate(bidx, bcnt, g).reshape(_ROWS, _D)
        g = _layers[i](spart, g, C[i], dis, W1[i])
    return g


# speedup vs baseline: 3.4659x; 1.0760x over previous
import functools
import math

import jax
import jax.numpy as jnp
from jax import lax
from jax.experimental import pallas as pl
from jax.experimental.pallas import tpu as pltpu
from jax.experimental.pallas import tpu_sc as plsc

_N = 10000
_E = 160000
_D = 256
_NL = 4
_ALPHA = 0.5

_NT = 32
_K = 128
_KB = 1024
_EPAD = 163840
_NSC = _EPAD // _KB
_RANGE = 320
_ROWS = _NT * _RANGE
_B = 64
_CAPB = _EPAD // _B + 4
_CAPW = _CAPB * 128
_TRASH = _RANGE
_NIB = 112

_BLK = 200
_NBLK = _N // _BLK

_sc_mesh = plsc.VectorSubcoreMesh(core_axis_name="c", subcore_axis_name="s")
_sc_params = pltpu.CompilerParams(needs_layout_passes=False)


@functools.partial(
    pl.kernel,
    out_type=(
        jax.ShapeDtypeStruct((_NT * _CAPW,), jnp.int32),
        jax.ShapeDtypeStruct((_NT, 16), jnp.int32),
        jax.ShapeDtypeStruct((_ROWS, 16), jnp.float32),
    ),
    mesh=_sc_mesh,
    compiler_params=_sc_params,
    scratch_types=[
        pltpu.VMEM((2 * _KB,), jnp.int32),
        pltpu.VMEM((2 * _KB,), jnp.int32),
        pltpu.VMEM((144,), jnp.int32),
        pltpu.VMEM((144,), jnp.int32),
        pltpu.VMEM((16,), jnp.int32),
        pltpu.VMEM((_RANGE + 1, 16), jnp.float32),
        pltpu.SemaphoreType.DMA,
        pltpu.SemaphoreType.DMA,
    ],
)
def _bucketize(eint_hbm, bidx_out, cnt_out, deg_out,
               ebuf0, ebuf1, srow, slc, nvb, dacc, semA, semB):
    c = lax.axis_index("c")
    s = lax.axis_index("s")
    w = c * 16 + s
    obase = w * _CAPW
    lane = lax.iota(jnp.int32, 16)
    lo = w * _RANGE

    def _fire(t, ebuf, sem):
        pltpu.async_copy(eint_hbm.at[pl.ds(t * 2 * _KB, 2 * _KB)], ebuf, sem)

    def _waitdma(t, ebuf, sem):
        pltpu.make_async_copy(eint_hbm.at[pl.ds(t * 2 * _KB, 2 * _KB)],
                              ebuf, sem).wait()

    def _scanbuf(ebuf, carry):
        def _grp(u, carry2):
            nf2, cnt2 = carry2
            rv = ebuf[pl.ds(u * 16, 16)]
            cv = ebuf[pl.ds(_KB + u * 16, 16)] - lo
            m = (cv >= 0) & (cv < _RANGE)
            mi = m.astype(jnp.int32)
            rank = plsc.cumsum(mi) - 1
            nm = jnp.sum(mi)
            off = jnp.where(m, nf2 + rank, 128 + lane)
            plsc.store_scatter(srow, [off], rv)
            plsc.store_scatter(slc, [off], cv)
            nf3 = nf2 + nm

            def _flush():
                base = obase + cnt2 * 128
                pltpu.sync_copy(srow.at[pl.ds(0, _B)],
                                bidx_out.at[pl.ds(base, _B)])
                pltpu.sync_copy(slc.at[pl.ds(0, _B)],
                                bidx_out.at[pl.ds(base + _B, _B)])
                srow[pl.ds(0, 16)] = srow[pl.ds(_B, 16)]
                slc[pl.ds(0, 16)] = slc[pl.ds(_B, 16)]

            pl.when(nf3 >= _B)(_flush)
            hit = nf3 >= _B
            return (jnp.where(hit, nf3 - _B, nf3),
                    jnp.where(hit, cnt2 + 1, cnt2))

        return lax.fori_loop(0, _KB // 16, _grp, carry)

    _fire(0, ebuf0, semA)

    def _scan2(tt, carry):
        t0 = tt * 2
        t1 = t0 + 1
        _fire(t1, ebuf1, semB)
        _waitdma(t0, ebuf0, semA)
        carry = _scanbuf(ebuf0, carry)

        @pl.when(t0 + 2 < _NSC)
        def _():
            _fire(t0 + 2, ebuf0, semA)

        _waitdma(t1, ebuf1, semB)
        return _scanbuf(ebuf1, carry)

    nf, cnt = lax.fori_loop(0, _NSC // 2, _scan2,
                            (jnp.int32(0), jnp.int32(0)))

    zero16 = jnp.zeros((16,), jnp.int32)
    trash16 = jnp.full((16,), _TRASH, jnp.int32)
    for t in range(5):
        plsc.store_scatter(srow, [nf + t * 16 + lane], zero16)
        plsc.store_scatter(slc, [nf + t * 16 + lane], trash16)
    base = obase + cnt * 128
    pltpu.sync_copy(srow.at[pl.ds(0, _B)], bidx_out.at[pl.ds(base, _B)])
    pltpu.sync_copy(slc.at[pl.ds(0, _B)], bidx_out.at[pl.ds(base + _B, _B)])
    nch = cnt + jnp.where(nf > 0, 1, 0).astype(jnp.int32)
    nvb[...] = jnp.broadcast_to(nch, (16,)).astype(jnp.int32)
    pltpu.sync_copy(nvb, cnt_out.at[w])

    fz = jnp.zeros((16,), jnp.float32)

    def _dz(i, carry):
        dacc[i, :] = fz
        return carry

    lax.fori_loop(0, _RANGE + 1, _dz, 0)
    one16 = jnp.ones((16,), jnp.float32)

    def _dchunk(j, carry):
        pltpu.sync_copy(bidx_out.at[pl.ds(obase + j * 128 + _B, _B)],
                        ebuf0.at[pl.ds(0, _B)])
        for u in range(_B // 16):
            lcv = ebuf0[pl.ds(u * 16, 16)]
            for e2 in range(16):
                lc = jnp.sum(jnp.where(lane == e2, lcv, 0))
                plsc.addupdate(dacc.at[lc], one16)
        return carry

    lax.fori_loop(0, nch, _dchunk, 0)
    pltpu.sync_copy(dacc.at[pl.ds(0, _RANGE)],
                    deg_out.at[pl.ds(w * _RANGE, _RANGE)])


@functools.partial(
    pl.kernel,
    out_type=jax.ShapeDtypeStruct((_ROWS * _D,), jnp.float32),
    mesh=_sc_mesh,
    compiler_params=_sc_params,
    scratch_types=[
        pltpu.VMEM((16,), jnp.int32),
        pltpu.VMEM((_NIB * 128,), jnp.int32),
        pltpu.VMEM((_B, _D), jnp.float32),
        pltpu.VMEM((_B, _D), jnp.float32),
        pltpu.VMEM(((_RANGE + 1) * _D,), jnp.float32),
        pltpu.SemaphoreType.DMA,
        pltpu.SemaphoreType.DMA,
    ],
)
def _propagate(bidx_hbm, cnt_hbm, g_hbm, s_out,
               nv, ibig, buf0, buf1, acc, sem0, sem1):
    c = lax.axis_index("c")
    s = lax.axis_index("s")
    w = c * 16 + s
    obase = w * _CAPW
    lane = lax.iota(jnp.int32, 16)

    pltpu.sync_copy(cnt_hbm.at[w], nv)
    n = jnp.sum(jnp.where(lane == 0, nv[...], 0))
    pltpu.sync_copy(bidx_hbm.at[pl.ds(obase, _NIB * 128)], ibig)

    def _slot(j):
        return (j % _NIB) * 128

    def _stage(j):
        @pl.when(j >= _NIB)
        def _():
            pltpu.sync_copy(bidx_hbm.at[pl.ds(obase + j * 128, 128)],
                            ibig.at[pl.ds(_slot(j), 128)])

    def _fire(j, buf, sem):
        pltpu.async_copy(g_hbm.at[ibig.at[pl.ds(_slot(j), _B)]], buf, sem)

    def _wait(j, buf, sem):
        pltpu.make_async_copy(g_hbm.at[ibig.at[pl.ds(_slot(j), _B)]],
                              buf, sem).wait()

    def _compute(j, buf):
        base = _slot(j) + _B
        for u in range(_B // 16):
            lcv = ibig[pl.ds(base + u * 16, 16)]
            for e2 in range(16):
                lcb = lcv.at[jnp.full((16,), e2, jnp.int32)].get(
                    mode="promise_in_bounds")
                tgt = lcb * _D + lane
                e = u * 16 + e2
                for d in range(_D // 16):
                    plsc.addupdate_scatter(acc, [tgt + d * 16],
                                           buf[e, pl.ds(d * 16, 16)])

    @pl.when(n > 0)
    def _prime():
        _fire(0, buf0, sem0)

    fz = jnp.zeros((16,), jnp.float32)

    def _az(i, carry):
        acc[pl.ds(i * 16, 16)] = fz
        return carry

    lax.fori_loop(0, (_RANGE + 1) * _D // 16, _az, 0)

    def _body(jj, carry):
        j0 = jj * 2
        j1 = j0 + 1

        @pl.when(j1 < n)
        def _():
            _stage(j1)
            _fire(j1, buf1, sem1)

        _wait(j0, buf0, sem0)
        _compute(j0, buf0)

        @pl.when(j0 + 2 < n)
        def _():
            _stage(j0 + 2)
            _fire(j0 + 2, buf0, sem0)

        @pl.when(j1 < n)
        def _():
            _wait(j1, buf1, sem1)
            _compute(j1, buf1)

        return carry

    lax.fori_loop(0, (n + 1) // 2, _body, 0)
    pltpu.sync_copy(acc.at[pl.ds(0, _RANGE * _D)],
                    s_out.at[pl.ds(w * _RANGE * _D, _RANGE * _D)])


def _prologue_body(x_ref, wp_ref, bp_ref, w2_ref, deg_ref,
                   g_ref, c_ref, dis_ref):
    h0 = jnp.dot(x_ref[...], wp_ref[...],
                 preferred_element_type=jnp.float32) + bp_ref[...]
    dis = lax.rsqrt(deg_ref[...][:, 0:1] + 1.0)
    g_ref[...] = dis * h0
    dis_ref[...] = jnp.broadcast_to(dis, (_BLK, 128))
    for i in range(_NL):
        beta = math.log(1.0 / (i + 1) + 1.0)
        c_ref[i, :, :] = (_ALPHA * (1.0 - beta)) * h0 + (_ALPHA * beta) * jnp.dot(
            h0, w2_ref[i], preferred_element_type=jnp.float32)


_prologue = pl.pallas_call(
    _prologue_body,
    grid=(_NBLK,),
    in_specs=[
        pl.BlockSpec((_BLK, _D), lambda j: (j, 0)),
        pl.BlockSpec((_D, _D), lambda j: (0, 0)),
        pl.BlockSpec((1, _D), lambda j: (0, 0)),
        pl.BlockSpec((_NL, _D, _D), lambda j: (0, 0, 0)),
        pl.BlockSpec((_BLK, 16), lambda j: (j, 0)),
    ],
    out_specs=[
        pl.BlockSpec((_BLK, _D), lambda j: (j, 0)),
        pl.BlockSpec((_NL, _BLK, _D), lambda j: (0, j, 0)),
        pl.BlockSpec((_BLK, 128), lambda j: (j, 0)),
    ],
    out_shape=[
        jax.ShapeDtypeStruct((_N, _D), jnp.float32),
        jax.ShapeDtypeStruct((_NL, _N, _D), jnp.float32),
        jax.ShapeDtypeStruct((_N, 128), jnp.float32),
    ],
)


def _layer_body(beta, is_last, s_ref, g_ref, cc_ref, dis_ref, w1_ref, o_ref):
    dis = dis_ref[...][:, 0:1]
    xp = dis * (s_ref[...] + g_ref[...])
    x1 = _ALPHA * xp
    out = cc_ref[...] + (1.0 - beta) * x1 + beta * jnp.dot(
        x1, w1_ref[...], preferred_element_type=jnp.float32)
    h = jnp.maximum(out, 0.0)
    o_ref[...] = h if is_last else dis * h


def _make_layer(i):
    beta = math.log(1.0 / (i + 1) + 1.0)
    return pl.pallas_call(
        functools.partial(_layer_body, beta, i == _NL - 1),
        grid=(_NBLK,),
        in_specs=[
            pl.BlockSpec((_BLK, _D), lambda j: (j, 0)),
            pl.BlockSpec((_BLK, _D), lambda j: (j, 0)),
            pl.BlockSpec((_BLK, _D), lambda j: (j, 0)),
            pl.BlockSpec((_BLK, 128), lambda j: (j, 0)),
            pl.BlockSpec((_D, _D), lambda j: (0, 0)),
        ],
        out_specs=pl.BlockSpec((_BLK, _D), lambda j: (j, 0)),
        out_shape=jax.ShapeDtypeStruct((_N, _D), jnp.float32),
    )


_layers = [_make_layer(i) for i in range(_NL)]


def kernel(x, edge_index, Wp, bp, W1, W2):
    row = edge_index[0]
    col = edge_index[1]
    pad = _EPAD - _E
    rowp = jnp.concatenate([row, jnp.zeros((pad,), row.dtype)])
    colp = jnp.concatenate([col, jnp.full((pad,), _ROWS, col.dtype)])
    eint = jnp.concatenate([rowp.reshape(_NSC, _KB),
                            colp.reshape(_NSC, _KB)], axis=1).reshape(-1)

    bidx, bcnt, deg = _bucketize(eint)
    g, C, dis = _prologue(x, Wp, bp.reshape(1, _D), W2, deg[:_N])

    for i in range(_NL):
        spart = _propagate(bidx, bcnt, g).reshape(_ROWS, _D)
        g = _layers[i](spart, g, C[i], dis, W1[i])
    return g
